# baseline (speedup 1.0000x reference)
import functools

import jax
import jax.numpy as jnp
from jax import lax
from jax.experimental import pallas as pl
from jax.experimental.pallas import tpu as pltpu
from jax.experimental.pallas import tpu_sc as plsc

_NC, _NS = 2, 16
_NW = _NC * _NS
_R, _D = 16384, 4096
_ROWS_W = _R // _NW


def _ring(x_hbm, o_hbm, base, ch, n, slots, srs, sws):
    def rd(i, b):
        return pltpu.make_async_copy(
            x_hbm.at[pl.ds(base + i * ch, ch)], slots[b], srs[b])

    def wr(i, b):
        return pltpu.make_async_copy(
            slots[b], o_hbm.at[pl.ds(base + i * ch, ch)], sws[b])

    rd(0, 0).start()
    rd(1, 1).start()
    for i in range(n):
        b = i % 2
        rd(i, b).wait()
        wr(i, b).start()
        if i + 2 <= n - 1:
            wr(i, b).wait()
            rd(i + 2, b).start()
    wr(n - 2, (n - 2) % 2).wait()
    wr(n - 1, (n - 1) % 2).wait()


def _sc_copy_body(x_hbm, o_hbm, sp_a, sp_b, sr0, sr1, sw0, sw1):
    c = lax.axis_index("c")
    s = lax.axis_index("s")
    wid = s * _NC + c
    base = wid * _ROWS_W
    srs, sws = (sr0, sr1), (sw0, sw1)

    @pl.when(s < _NS - 1)
    def _():
        _ring(x_hbm, o_hbm, base, 16, _ROWS_W // 16,
              (sp_a.at[s], sp_b.at[s]), srs, sws)

    @pl.when(s == _NS - 1)
    def _():
        _ring(x_hbm, o_hbm, base, 8, _ROWS_W // 8,
              (sp_a.at[_NS - 1, pl.ds(0, 8)], sp_a.at[_NS - 1, pl.ds(8, 8)]),
              srs, sws)


def kernel(x, score):
    del score
    B, S, D = x.shape
    x2 = x.reshape(_R, _D)
    mesh = plsc.VectorSubcoreMesh(core_axis_name="c", subcore_axis_name="s")
    f = functools.partial(
        pl.kernel,
        out_type=jax.ShapeDtypeStruct((_R, _D), x.dtype),
        mesh=mesh,
        scratch_types=[
            pltpu.VMEM_SHARED((_NS, 16, _D), jnp.float32),
            pltpu.VMEM_SHARED((_NS - 1, 16, _D), jnp.float32),
            pltpu.SemaphoreType.DMA,
            pltpu.SemaphoreType.DMA,
            pltpu.SemaphoreType.DMA,
            pltpu.SemaphoreType.DMA,
        ],
    )(_sc_copy_body)
    out = f(x2)
    return out.reshape(B, S, D)
